# baseline (device time: 105646 ns/iter reference)
import jax
import jax.numpy as jnp
from jax import lax
from jax.experimental import pallas as pl
from jax.experimental.pallas import tpu as pltpu

N_DEV = 16
M = 1024
K = 512
N = 512
CHUNK = M // N_DEV
N_HOPS = 2 * (N_DEV - 1)


def kernel(t, W):
    def body(t_ref, w_ref, out_ref, comm_ref, send_sems, recv_sems):
        i = lax.axis_index("i")
        left = lax.rem(i + N_DEV - 1, N_DEV)
        right = lax.rem(i + 1, N_DEV)

        barrier_sem = pltpu.get_barrier_semaphore()
        for nbr in (left, right):
            pl.semaphore_signal(
                barrier_sem, inc=1,
                device_id=(nbr,), device_id_type=pl.DeviceIdType.MESH,
            )
        pl.semaphore_wait(barrier_sem, 2)

        out_ref[...] = jnp.dot(
            t_ref[...], w_ref[...], preferred_element_type=jnp.float32
        )

        for s in range(N_DEV - 1):
            c_send = lax.rem(i - s + N_DEV, N_DEV)
            rdma = pltpu.make_async_remote_copy(
                src_ref=out_ref.at[pl.ds(c_send * CHUNK, CHUNK), :],
                dst_ref=comm_ref.at[s],
                send_sem=send_sems.at[s],
                recv_sem=recv_sems.at[s],
                device_id=(right,),
                device_id_type=pl.DeviceIdType.MESH,
            )
            rdma.start()
            rdma.wait()
            c_recv = lax.rem(i - s - 1 + N_DEV, N_DEV)
            out_ref[pl.ds(c_recv * CHUNK, CHUNK), :] += comm_ref[s]

        for s in range(N_DEV - 1):
            h = N_DEV - 1 + s
            c_send = lax.rem(i + 1 - s + N_DEV, N_DEV)
            rdma = pltpu.make_async_remote_copy(
                src_ref=out_ref.at[pl.ds(c_send * CHUNK, CHUNK), :],
                dst_ref=comm_ref.at[h],
                send_sem=send_sems.at[h],
                recv_sem=recv_sems.at[h],
                device_id=(right,),
                device_id_type=pl.DeviceIdType.MESH,
            )
            rdma.start()
            rdma.wait()
            c_recv = lax.rem(i - s + N_DEV, N_DEV)
            out_ref[pl.ds(c_recv * CHUNK, CHUNK), :] = comm_ref[h]

    return pl.pallas_call(
        body,
        out_shape=jax.ShapeDtypeStruct((M, N), jnp.float32),
        in_specs=[
            pl.BlockSpec(memory_space=pltpu.VMEM),
            pl.BlockSpec(memory_space=pltpu.VMEM),
        ],
        out_specs=pl.BlockSpec(memory_space=pltpu.VMEM),
        scratch_shapes=[
            pltpu.VMEM((N_HOPS, CHUNK, N), jnp.float32),
            pltpu.SemaphoreType.DMA((N_HOPS,)),
            pltpu.SemaphoreType.DMA((N_HOPS,)),
        ],
        compiler_params=pltpu.CompilerParams(collective_id=0),
    )(t, W)


# device time: 51826 ns/iter; 2.0385x vs baseline; 2.0385x over previous
import jax
import jax.numpy as jnp
from jax import lax
from jax.experimental import pallas as pl
from jax.experimental.pallas import tpu as pltpu

N_DEV = 16
M = 1024
K = 512
N = 512

S = M // 2
STREAMS = ((0, (1, 2, 4, 8)), (S, (4, 8, 1, 2)))
N_EXCH = 16
COMM_ROWS = 2 * 2 * (256 + 128 + 64 + 32)


def kernel(t, W):
    def body(t_ref, w_ref, out_ref, comm_ref, send_sems, recv_sems):
        i = lax.axis_index("i")

        barrier_sem = pltpu.get_barrier_semaphore()
        for m in (1, 2, 4, 8):
            pl.semaphore_signal(
                barrier_sem, inc=1,
                device_id=(jnp.bitwise_xor(i, m),),
                device_id_type=pl.DeviceIdType.MESH,
            )

        out_ref[...] = jnp.dot(
            t_ref[...], w_ref[...], preferred_element_type=jnp.float32
        )

        pl.semaphore_wait(barrier_sem, 4)

        def bit_of(mask):
            return lax.rem(lax.div(i, mask), 2)

        lo = [jnp.int32(base) for base, _ in STREAMS]
        comm_off = 0
        sem_idx = 0

        for j in range(4):
            H = (S // 2) >> j
            started = []
            for s, (_, masks) in enumerate(STREAMS):
                m = masks[j]
                b = bit_of(m)
                send_lo = lo[s] + (1 - b) * H
                keep_lo = lo[s] + b * H
                rdma = pltpu.make_async_remote_copy(
                    src_ref=out_ref.at[pl.ds(send_lo, H), :],
                    dst_ref=comm_ref.at[pl.ds(comm_off, H), :],
                    send_sem=send_sems.at[sem_idx],
                    recv_sem=recv_sems.at[sem_idx],
                    device_id=(jnp.bitwise_xor(i, m),),
                    device_id_type=pl.DeviceIdType.MESH,
                )
                rdma.start()
                started.append((rdma, keep_lo, comm_off, H))
                lo[s] = keep_lo
                comm_off += H
                sem_idx += 1
            for rdma, keep_lo, off, H in started:
                rdma.wait()
                out_ref[pl.ds(keep_lo, H), :] += comm_ref[pl.ds(off, H), :]

        for j in range(4):
            H = (S // 16) << j
            started = []
            for s, (_, masks) in enumerate(STREAMS):
                m = masks[3 - j]
                b = bit_of(m)
                base_lo = lo[s] - b * H
                partner_lo = base_lo + (1 - b) * H
                rdma = pltpu.make_async_remote_copy(
                    src_ref=out_ref.at[pl.ds(lo[s], H), :],
                    dst_ref=comm_ref.at[pl.ds(comm_off, H), :],
                    send_sem=send_sems.at[sem_idx],
                    recv_sem=recv_sems.at[sem_idx],
                    device_id=(jnp.bitwise_xor(i, m),),
                    device_id_type=pl.DeviceIdType.MESH,
                )
                rdma.start()
                started.append((rdma, partner_lo, comm_off, H))
                lo[s] = base_lo
                comm_off += H
                sem_idx += 1
            for rdma, partner_lo, off, H in started:
                rdma.wait()
                out_ref[pl.ds(partner_lo, H), :] = comm_ref[pl.ds(off, H), :]

    return pl.pallas_call(
        body,
        out_shape=jax.ShapeDtypeStruct((M, N), jnp.float32),
        in_specs=[
            pl.BlockSpec(memory_space=pltpu.VMEM),
            pl.BlockSpec(memory_space=pltpu.VMEM),
        ],
        out_specs=pl.BlockSpec(memory_space=pltpu.VMEM),
        scratch_shapes=[
            pltpu.VMEM((COMM_ROWS, N), jnp.float32),
            pltpu.SemaphoreType.DMA((N_EXCH,)),
            pltpu.SemaphoreType.DMA((N_EXCH,)),
        ],
        compiler_params=pltpu.CompilerParams(collective_id=0),
    )(t, W)


# device time: 48531 ns/iter; 2.1769x vs baseline; 1.0679x over previous
import jax
import jax.numpy as jnp
from jax import lax
from jax.experimental import pallas as pl
from jax.experimental.pallas import tpu as pltpu

N_DEV = 16
M = 1024
K = 512
N = 512

S = M // 2
STREAMS = ((0, (1, 2, 4, 8)), (S, (4, 8, 1, 2)))
N_EXCH = 16
COMM_ROWS = 2 * (256 + 128 + 64 + 32)


def kernel(t, W):
    def body(t_ref, w_ref, out_ref, comm_ref, send_sems, recv_sems):
        i = lax.axis_index("i")

        barrier_sem = pltpu.get_barrier_semaphore()
        for m in (1, 2, 4, 8):
            pl.semaphore_signal(
                barrier_sem, inc=1,
                device_id=(jnp.bitwise_xor(i, m),),
                device_id_type=pl.DeviceIdType.MESH,
            )

        out_ref[...] = jnp.dot(
            t_ref[...], w_ref[...], preferred_element_type=jnp.float32
        )

        pl.semaphore_wait(barrier_sem, 4)

        def bit_of(mask):
            return lax.rem(lax.div(i, mask), 2)

        lo = [jnp.int32(base) for base, _ in STREAMS]
        pending = [None, None]
        ctr = {"sem": 0, "comm": 0}

        def start(s, slot):
            _, masks = STREAMS[s]
            sem = ctr["sem"]
            ctr["sem"] += 1
            if slot < 4:
                H = (S // 2) >> slot
                m = masks[slot]
                b = bit_of(m)
                send_lo = lo[s] + (1 - b) * H
                keep_lo = lo[s] + b * H
                off = ctr["comm"]
                ctr["comm"] += H
                rdma = pltpu.make_async_remote_copy(
                    src_ref=out_ref.at[pl.ds(send_lo, H), :],
                    dst_ref=comm_ref.at[pl.ds(off, H), :],
                    send_sem=send_sems.at[sem],
                    recv_sem=recv_sems.at[sem],
                    device_id=(jnp.bitwise_xor(i, m),),
                    device_id_type=pl.DeviceIdType.MESH,
                )
                rdma.start()
                lo[s] = keep_lo
                pending[s] = (rdma, keep_lo, off, H)
            else:
                H = (S // 16) << (slot - 4)
                m = masks[7 - slot]
                b = bit_of(m)
                rdma = pltpu.make_async_remote_copy(
                    src_ref=out_ref.at[pl.ds(lo[s], H), :],
                    dst_ref=out_ref.at[pl.ds(lo[s], H), :],
                    send_sem=send_sems.at[sem],
                    recv_sem=recv_sems.at[sem],
                    device_id=(jnp.bitwise_xor(i, m),),
                    device_id_type=pl.DeviceIdType.MESH,
                )
                rdma.start()
                lo[s] = lo[s] - b * H
                pending[s] = (rdma, None, None, None)

        def finish(s):
            rdma, keep_lo, off, H = pending[s]
            rdma.wait()
            if off is not None:
                out_ref[pl.ds(keep_lo, H), :] += comm_ref[pl.ds(off, H), :]

        for s in range(2):
            start(s, 0)
        for slot in range(1, 8):
            for s in range(2):
                finish(s)
                start(s, slot)
        for s in range(2):
            finish(s)

    return pl.pallas_call(
        body,
        out_shape=jax.ShapeDtypeStruct((M, N), jnp.float32),
        in_specs=[
            pl.BlockSpec(memory_space=pltpu.VMEM),
            pl.BlockSpec(memory_space=pltpu.VMEM),
        ],
        out_specs=pl.BlockSpec(memory_space=pltpu.VMEM),
        scratch_shapes=[
            pltpu.VMEM((COMM_ROWS, N), jnp.float32),
            pltpu.SemaphoreType.DMA((N_EXCH,)),
            pltpu.SemaphoreType.DMA((N_EXCH,)),
        ],
        compiler_params=pltpu.CompilerParams(collective_id=0),
    )(t, W)


# device time: 36299 ns/iter; 2.9104x vs baseline; 1.3370x over previous
import jax
import jax.numpy as jnp
from jax import lax
from jax.experimental import pallas as pl
from jax.experimental.pallas import tpu as pltpu

N_DEV = 16
M = 1024
K = 512
N = 512

S = M // 2
STREAMS = ((0, (1, 2, 4, 8)), (S, (4, 8, 1, 2)))
N_EXCH = 16
RS_ROWS = 2 * (256 + 128 + 64 + 32)


def kernel(t, W):
    def body(t_ref, w_ref, out_ref, stage_ref, comm_ref, ag_ref,
             send_sems, recv_sems):
        i = lax.axis_index("i")

        barrier_sem = pltpu.get_barrier_semaphore()
        for m in (1, 2, 4, 8):
            pl.semaphore_signal(
                barrier_sem, inc=1,
                device_id=(jnp.bitwise_xor(i, m),),
                device_id_type=pl.DeviceIdType.MESH,
            )

        out_ref[...] = jnp.dot(
            t_ref[...], w_ref[...], preferred_element_type=jnp.float32
        )

        pl.semaphore_wait(barrier_sem, 4)

        def bit_of(mask):
            return lax.rem(lax.div(i, mask), 2)

        lo = [jnp.int32(base) for base, _ in STREAMS]
        pending = [None, None]
        ctr = {"sem": 0, "comm": 0}

        def start(s, slot):
            _, masks = STREAMS[s]
            sem = ctr["sem"]
            ctr["sem"] += 1
            if slot < 4:
                H = (S // 2) >> slot
                m = masks[slot]
                b = bit_of(m)
                send_lo = lo[s] + (1 - b) * H
                keep_lo = lo[s] + b * H
                off = ctr["comm"]
                ctr["comm"] += H
                stage_ref[pl.ds(off, H), :] = out_ref[
                    pl.ds(send_lo, H), :
                ].astype(jnp.bfloat16)
                rdma = pltpu.make_async_remote_copy(
                    src_ref=stage_ref.at[pl.ds(off, H), :],
                    dst_ref=comm_ref.at[pl.ds(off, H), :],
                    send_sem=send_sems.at[sem],
                    recv_sem=recv_sems.at[sem],
                    device_id=(jnp.bitwise_xor(i, m),),
                    device_id_type=pl.DeviceIdType.MESH,
                )
                rdma.start()
                lo[s] = keep_lo
                pending[s] = (rdma, keep_lo, off, H)
            else:
                H = (S // 16) << (slot - 4)
                m = masks[7 - slot]
                b = bit_of(m)
                rdma = pltpu.make_async_remote_copy(
                    src_ref=ag_ref.at[pl.ds(lo[s], H), :],
                    dst_ref=ag_ref.at[pl.ds(lo[s], H), :],
                    send_sem=send_sems.at[sem],
                    recv_sem=recv_sems.at[sem],
                    device_id=(jnp.bitwise_xor(i, m),),
                    device_id_type=pl.DeviceIdType.MESH,
                )
                rdma.start()
                lo[s] = lo[s] - b * H
                pending[s] = (rdma, None, None, None)

        def finish(s, slot):
            rdma, keep_lo, off, H = pending[s]
            rdma.wait()
            if off is not None:
                out_ref[pl.ds(keep_lo, H), :] += comm_ref[
                    pl.ds(off, H), :
                ].astype(jnp.float32)
                if slot == 3:
                    ag_ref[pl.ds(keep_lo, H), :] = out_ref[
                        pl.ds(keep_lo, H), :
                    ].astype(jnp.bfloat16)

        for s in range(2):
            start(s, 0)
        for slot in range(1, 8):
            for s in range(2):
                finish(s, slot - 1)
                start(s, slot)
        for s in range(2):
            finish(s, 7)

        out_ref[...] = ag_ref[...].astype(jnp.float32)

    return pl.pallas_call(
        body,
        out_shape=jax.ShapeDtypeStruct((M, N), jnp.float32),
        in_specs=[
            pl.BlockSpec(memory_space=pltpu.VMEM),
            pl.BlockSpec(memory_space=pltpu.VMEM),
        ],
        out_specs=pl.BlockSpec(memory_space=pltpu.VMEM),
        scratch_shapes=[
            pltpu.VMEM((RS_ROWS, N), jnp.bfloat16),
            pltpu.VMEM((RS_ROWS, N), jnp.bfloat16),
            pltpu.VMEM((M, N), jnp.bfloat16),
            pltpu.SemaphoreType.DMA((N_EXCH,)),
            pltpu.SemaphoreType.DMA((N_EXCH,)),
        ],
        compiler_params=pltpu.CompilerParams(collective_id=0),
    )(t, W)


# device time: 28077 ns/iter; 3.7627x vs baseline; 1.2928x over previous
import jax
import jax.numpy as jnp
from jax import lax
from jax.experimental import pallas as pl
from jax.experimental.pallas import tpu as pltpu

N_DEV = 16
M = 1024
K = 512
N = 512

STREAMS = ((0, 640, (1, 4)), (640, 384, (4, 1)))
N_EXCH = 24
RS_ROWS = 3 * (160 + 40 + 96 + 24)


def kernel(t, W):
    def body(t_ref, w_ref, out_ref, stage_ref, comm_ref, ag_ref,
             send_sems, recv_sems):
        i = lax.axis_index("i")

        def group(u):
            g = lax.rem(lax.div(i, u), 4)
            return g, i - g * u

        barrier_sem = pltpu.get_barrier_semaphore()
        for u in (1, 4):
            g, gbase = group(u)
            for d in (1, 2, 3):
                peer = gbase + lax.rem(g + d, 4) * u
                pl.semaphore_signal(
                    barrier_sem, inc=1,
                    device_id=(peer,),
                    device_id_type=pl.DeviceIdType.MESH,
                )

        out_ref[...] = jnp.dot(
            t_ref[...], w_ref[...], preferred_element_type=jnp.float32
        )

        pl.semaphore_wait(barrier_sem, 6)

        lo = [jnp.int32(base) for base, _, _ in STREAMS]
        pending = [None, None]
        ctr = {"sem": 0, "comm": 0, "stage": 0}

        def start(s, slot):
            _, R, units = STREAMS[s]
            sem_base = ctr["sem"]
            ctr["sem"] += 3
            if slot < 2:
                H = R // 4 if slot == 0 else R // 16
                u = units[slot]
                g, gbase = group(u)
                keep_lo = lo[s] + g * H
                comm_base = ctr["comm"]
                ctr["comm"] += 3 * H
                rdmas = []
                for d in (1, 2, 3):
                    jm = lax.rem(g + d, 4)
                    peer = gbase + jm * u
                    src_off = ctr["stage"]
                    ctr["stage"] += H
                    stage_ref[pl.ds(src_off, H), :] = out_ref[
                        pl.ds(lo[s] + jm * H, H), :
                    ].astype(jnp.bfloat16)
                    r = 4 - d
                    rdma = pltpu.make_async_remote_copy(
                        src_ref=stage_ref.at[pl.ds(src_off, H), :],
                        dst_ref=comm_ref.at[
                            pl.ds(comm_base + (r - 1) * H, H), :
                        ],
                        send_sem=send_sems.at[sem_base + r - 1],
                        recv_sem=recv_sems.at[sem_base + r - 1],
                        device_id=(peer,),
                        device_id_type=pl.DeviceIdType.MESH,
                    )
                    rdma.start()
                    rdmas.append(rdma)
                lo[s] = keep_lo
                pending[s] = (rdmas, keep_lo, comm_base, slot, H)
            else:
                H = R // 16 if slot == 2 else R // 4
                u = units[3 - slot]
                g, gbase = group(u)
                rdmas = []
                for d in (1, 2, 3):
                    peer = gbase + lax.rem(g + d, 4) * u
                    r = 4 - d
                    rdma = pltpu.make_async_remote_copy(
                        src_ref=ag_ref.at[pl.ds(lo[s], H), :],
                        dst_ref=ag_ref.at[pl.ds(lo[s], H), :],
                        send_sem=send_sems.at[sem_base + r - 1],
                        recv_sem=recv_sems.at[sem_base + r - 1],
                        device_id=(peer,),
                        device_id_type=pl.DeviceIdType.MESH,
                    )
                    rdma.start()
                    rdmas.append(rdma)
                lo[s] = lo[s] - g * H
                pending[s] = (rdmas, None, None, slot, H)

        def finish(s):
            rdmas, keep_lo, comm_base, slot, H = pending[s]
            for rdma in rdmas:
                rdma.wait()
            if comm_base is not None:
                out_ref[pl.ds(keep_lo, H), :] += (
                    comm_ref[pl.ds(comm_base, H), :].astype(jnp.float32)
                    + comm_ref[pl.ds(comm_base + H, H), :].astype(
                        jnp.float32
                    )
                    + comm_ref[pl.ds(comm_base + 2 * H, H), :].astype(
                        jnp.float32
                    )
                )
                if slot == 1:
                    ag_ref[pl.ds(keep_lo, H), :] = out_ref[
                        pl.ds(keep_lo, H), :
                    ].astype(jnp.bfloat16)

        for s in range(2):
            start(s, 0)
        for slot in range(1, 4):
            for s in range(2):
                finish(s)
                start(s, slot)
        for s in range(2):
            finish(s)

        out_ref[...] = ag_ref[...].astype(jnp.float32)

    return pl.pallas_call(
        body,
        out_shape=jax.ShapeDtypeStruct((M, N), jnp.float32),
        in_specs=[
            pl.BlockSpec(memory_space=pltpu.VMEM),
            pl.BlockSpec(memory_space=pltpu.VMEM),
        ],
        out_specs=pl.BlockSpec(memory_space=pltpu.VMEM),
        scratch_shapes=[
            pltpu.VMEM((RS_ROWS, N), jnp.bfloat16),
            pltpu.VMEM((RS_ROWS, N), jnp.bfloat16),
            pltpu.VMEM((M, N), jnp.bfloat16),
            pltpu.SemaphoreType.DMA((N_EXCH,)),
            pltpu.SemaphoreType.DMA((N_EXCH,)),
        ],
        compiler_params=pltpu.CompilerParams(collective_id=0),
    )(t, W)
